# baseline (speedup 1.0000x reference)
import functools

import jax
import jax.numpy as jnp
from jax import lax
from jax.experimental import pallas as pl
from jax.experimental.pallas import tpu as pltpu
from jax.experimental.pallas import tpu_sc as plsc


def kernel(x, table):
    seq = x.shape[1]
    emb = table.shape[1]
    info = plsc.get_sparse_core_info()
    nw = info.num_cores * info.num_subcores
    rows_per_w = seq // nw
    mesh = plsc.VectorSubcoreMesh(core_axis_name="c", subcore_axis_name="s")

    @functools.partial(
        pl.kernel,
        out_type=jax.ShapeDtypeStruct((seq, emb), table.dtype),
        mesh=mesh,
    )
    def sc_copy(table_hbm, out_hbm):
        wid = lax.axis_index("s") * info.num_cores + lax.axis_index("c")
        base = wid * rows_per_w
        pltpu.sync_copy(
            table_hbm.at[pl.ds(base, rows_per_w)],
            out_hbm.at[pl.ds(base, rows_per_w)],
        )

    return sc_copy(table)[None, :, :]


# speedup vs baseline: 17.1962x; 17.1962x over previous
import functools

import jax
import jax.numpy as jnp
from jax import lax
from jax.experimental import pallas as pl
from jax.experimental.pallas import tpu as pltpu
from jax.experimental.pallas import tpu_sc as plsc


def kernel(x, table):
    seq = x.shape[1]
    emb = table.shape[1]
    info = plsc.get_sparse_core_info()
    nw = info.num_cores * info.num_subcores
    rows_per_w = seq // nw
    chunk = 32
    nchunks = rows_per_w // chunk
    mesh = plsc.VectorSubcoreMesh(core_axis_name="c", subcore_axis_name="s")

    @functools.partial(
        pl.kernel,
        out_type=jax.ShapeDtypeStruct((seq, emb), table.dtype),
        mesh=mesh,
        scratch_types=[
            pltpu.VMEM((2, chunk, emb), jnp.float32),
            pltpu.SemaphoreType.DMA,
            pltpu.SemaphoreType.DMA,
        ],
    )
    def sc_copy(table_hbm, out_hbm, buf, in_sem, out_sem):
        wid = lax.axis_index("s") * info.num_cores + lax.axis_index("c")
        base = wid * rows_per_w

        def in_copy(i, slot):
            return pltpu.make_async_copy(
                table_hbm.at[pl.ds(base + i * chunk, chunk)], buf.at[slot], in_sem
            )

        def out_copy(i, slot):
            return pltpu.make_async_copy(
                buf.at[slot], out_hbm.at[pl.ds(base + i * chunk, chunk)], out_sem
            )

        in_copy(0, 0).start()

        def body(i, _):
            slot = lax.rem(i, 2)
            nxt = 1 - slot

            @pl.when(i + 1 < nchunks)
            def _():
                in_copy(i + 1, nxt).start()

            in_copy(i, slot).wait()

            @pl.when(i > 0)
            def _():
                out_copy(i - 1, 1 - slot).wait()

            out_copy(i, slot).start()
            return 0

        lax.fori_loop(0, nchunks, body, 0)
        out_copy(nchunks - 1, lax.rem(nchunks - 1, 2)).wait()

    return sc_copy(table)[None, :, :]
